# baseline (speedup 1.0000x reference)
import functools

import jax
import jax.numpy as jnp
from jax import lax
from jax.experimental import pallas as pl
from jax.experimental.pallas import tpu as pltpu, tpu_sc as plsc

N = 10000
D = 128
E = 320000

NC = 2
NS = 16
NW = NC * NS
CHUNK = 128
SUP = 8
NSUP = 10
CPT = SUP * NSUP
E_PAD = NW * CPT * CHUNK
N_PAD = 10240
RPT = N_PAD // NS

_mesh = plsc.VectorSubcoreMesh(core_axis_name="c", subcore_axis_name="s")


@functools.partial(
    pl.kernel,
    mesh=_mesh,
    out_type=jax.ShapeDtypeStruct((NC, N_PAD), jnp.float32),
    scratch_types=[
        pltpu.VMEM_SHARED((N_PAD,), jnp.float32),
        pltpu.VMEM((CPT, CHUNK), jnp.int32),
        pltpu.VMEM((CHUNK,), jnp.float32),
    ],
)
def _deg_kernel(dst_hbm, zeros_hbm, out_hbm, acc, dst_v, ones_v):
    cid = lax.axis_index("c")
    sid = lax.axis_index("s")
    wid = cid * NS + sid
    pltpu.sync_copy(zeros_hbm, acc.at[pl.ds(sid * RPT, RPT)])
    pltpu.sync_copy(dst_hbm.at[wid], dst_v)
    for j in range(CHUNK // 16):
        ones_v[pl.ds(j * 16, 16)] = jnp.ones((16,), jnp.float32)
    plsc.subcore_barrier()

    def body(j, carry):
        pltpu.sync_copy(ones_v, acc.at[dst_v.at[j]], add=True)
        return carry

    lax.fori_loop(0, CPT, body, 0)
    plsc.subcore_barrier()
    pltpu.sync_copy(acc.at[pl.ds(sid * RPT, RPT)],
                    out_hbm.at[cid, pl.ds(sid * RPT, RPT)])


@functools.partial(
    pl.kernel,
    mesh=_mesh,
    out_type=jax.ShapeDtypeStruct((NC, N_PAD, D), jnp.float32),
    scratch_types=[
        pltpu.VMEM_SHARED((N_PAD, D), jnp.float32),
        pltpu.VMEM((2, SUP, CHUNK), jnp.int32),
        pltpu.VMEM((2, SUP, CHUNK), jnp.int32),
        pltpu.VMEM((2, CHUNK, D), jnp.float32),
        pltpu.SemaphoreType.DMA,
        pltpu.SemaphoreType.DMA,
        pltpu.SemaphoreType.DMA,
        pltpu.SemaphoreType.DMA,
        pltpu.SemaphoreType.DMA,
        pltpu.SemaphoreType.DMA,
    ],
)
def _agg_kernel(h_hbm, src_hbm, dst_hbm, zeros_hbm, out_hbm,
                acc, src_v, dst_v, rows_v, isem0, isem1, gsem0, gsem1,
                ssem0, ssem1):
    cid = lax.axis_index("c")
    sid = lax.axis_index("s")
    wid = cid * NS + sid
    isems = (isem0, isem1)
    gsems = (gsem0, gsem1)
    ssems = (ssem0, ssem1)

    def _idx_cps(s, b):
        return (pltpu.make_async_copy(src_hbm.at[wid, pl.ds(s * SUP, SUP)],
                                      src_v.at[b], isems[b]),
                pltpu.make_async_copy(dst_hbm.at[wid, pl.ds(s * SUP, SUP)],
                                      dst_v.at[b], isems[b]))

    def _gather(p, b, c):
        return pltpu.make_async_copy(h_hbm.at[src_v.at[b, c]], rows_v.at[p],
                                     gsems[p])

    def _scatter(p, b, c):
        return pltpu.async_copy(rows_v.at[p], acc.at[dst_v.at[b, c]],
                                ssems[p], add=True)

    for cp in _idx_cps(0, 0):
        cp.start()
    pltpu.sync_copy(zeros_hbm, acc.at[pl.ds(sid * RPT, RPT), :])
    for cp in _idx_cps(0, 0):
        cp.wait()
    plsc.subcore_barrier()

    def body(g, carry):
        for b in range(2):
            s = 2 * g + b

            @pl.when(s + 1 < NSUP)
            def _(b=b, s=s):
                for cp in _idx_cps(s + 1, 1 - b):
                    cp.start()

            _gather(0, b, 0).start()
            _gather(1, b, 1).start()
            for c in range(0, SUP, 2):
                _gather(0, b, c).wait()
                s0 = _scatter(0, b, c)
                _gather(1, b, c + 1).wait()
                s1 = _scatter(1, b, c + 1)
                s0.wait()
                if c + 2 < SUP:
                    _gather(0, b, c + 2).start()
                s1.wait()
                if c + 3 < SUP:
                    _gather(1, b, c + 3).start()

            @pl.when(s + 1 < NSUP)
            def _(b=b, s=s):
                for cp in _idx_cps(s + 1, 1 - b):
                    cp.wait()
        return carry

    lax.fori_loop(0, NSUP // 2, body, 0)
    plsc.subcore_barrier()
    pltpu.sync_copy(acc.at[pl.ds(sid * RPT, RPT), :],
                    out_hbm.at[cid, pl.ds(sid * RPT, RPT), :])


BLK = 2048
_GRID = N_PAD // BLK


def _mm_body(x_ref, w_ref, h_ref):
    h_ref[...] = jnp.dot(x_ref[...], w_ref[...],
                         preferred_element_type=jnp.float32)


def _scale_body(d0_ref, d1_ref, h_ref, hp_ref, dis_ref):
    deg = d0_ref[...] + d1_ref[...] + 1.0
    dis = lax.rsqrt(deg)
    dis_ref[...] = dis
    hp_ref[...] = dis * h_ref[...]


def _tc2_body(p0_ref, p1_ref, hp_ref, dis_ref, b_ref, w_ref, out_ref):
    dis = dis_ref[...]
    z = dis * (p0_ref[...] + p1_ref[...] + hp_ref[...]) + b_ref[...]
    out_ref[...] = dis * jnp.dot(z, w_ref[...],
                                 preferred_element_type=jnp.float32)


def _tc3_body(q0_ref, q1_ref, hp_ref, dis_ref, b_ref, out_ref):
    out_ref[...] = (dis_ref[...] * (q0_ref[...] + q1_ref[...] + hp_ref[...])
                    + b_ref[...])


_row_spec = pl.BlockSpec((BLK, D), lambda i: (i, 0))
_vec_spec = pl.BlockSpec((BLK, 1), lambda i: (i, 0))
_w_spec = pl.BlockSpec((D, D), lambda i: (0, 0))
_b_spec = pl.BlockSpec((1, D), lambda i: (0, 0))
_p1_spec = pl.BlockSpec((BLK, D), lambda i: (i + _GRID, 0))
_d1_spec = pl.BlockSpec((BLK, 1), lambda i: (i + _GRID, 0))


def _mm(xp, W1):
    return pl.pallas_call(
        _mm_body,
        grid=(_GRID,),
        in_specs=[_row_spec, _w_spec],
        out_specs=_row_spec,
        out_shape=jax.ShapeDtypeStruct((N_PAD, D), jnp.float32),
    )(xp, W1)


def _scale(degr, h1):
    return pl.pallas_call(
        _scale_body,
        grid=(_GRID,),
        in_specs=[_vec_spec, _d1_spec, _row_spec],
        out_specs=[_row_spec, _vec_spec],
        out_shape=[jax.ShapeDtypeStruct((N_PAD, D), jnp.float32),
                   jax.ShapeDtypeStruct((N_PAD, 1), jnp.float32)],
    )(degr, degr, h1)


def _tc2(Pr, hp, dis, b, W2):
    return pl.pallas_call(
        _tc2_body,
        grid=(_GRID,),
        in_specs=[_row_spec, _p1_spec, _row_spec, _vec_spec, _b_spec,
                  _w_spec],
        out_specs=_row_spec,
        out_shape=jax.ShapeDtypeStruct((N_PAD, D), jnp.float32),
    )(Pr, Pr, hp, dis, b, W2)


def _tc3(Qr, hp, dis, b):
    return pl.pallas_call(
        _tc3_body,
        grid=(_GRID,),
        in_specs=[_row_spec, _p1_spec, _row_spec, _vec_spec, _b_spec],
        out_specs=_row_spec,
        out_shape=jax.ShapeDtypeStruct((N_PAD, D), jnp.float32),
    )(Qr, Qr, hp, dis, b)


def kernel(x, edge_index, W1, b1, W2, b2):
    src = edge_index[0].astype(jnp.int32)
    dst = edge_index[1].astype(jnp.int32)
    pad = E_PAD - E
    iota = jnp.arange(pad, dtype=jnp.int32)
    src_p = jnp.concatenate([src, iota % N]).reshape(NW, CPT, CHUNK)
    dst_p = jnp.concatenate([dst, N + iota % (N_PAD - N)]).reshape(
        NW, CPT, CHUNK)
    zeros1 = jnp.zeros((RPT,), jnp.float32)
    zeros2 = jnp.zeros((RPT, D), jnp.float32)
    xp = jnp.concatenate([x, jnp.zeros((N_PAD - N, D), jnp.float32)])

    degp = _deg_kernel(dst_p, zeros1)
    h1 = _mm(xp, W1)
    h1p, dis = _scale(degp.reshape(NC * N_PAD, 1), h1)

    P = _agg_kernel(h1p, src_p, dst_p, zeros2)
    h2p = _tc2(P.reshape(NC * N_PAD, D), h1p, dis, b1.reshape(1, D), W2)
    Q = _agg_kernel(h2p, src_p, dst_p, zeros2)
    out = _tc3(Q.reshape(NC * N_PAD, D), h2p, dis, b2.reshape(1, D))
    return out[:N]


# speedup vs baseline: 1.1944x; 1.1944x over previous
import functools

import jax
import jax.numpy as jnp
from jax import lax
from jax.experimental import pallas as pl
from jax.experimental.pallas import tpu as pltpu, tpu_sc as plsc

N = 10000
D = 128
E = 320000

NC = 2
NS = 16
NW = NC * NS
CHUNK = 128
SUP = 8
NSUP = 10
CPT = SUP * NSUP
E_PAD = NW * CPT * CHUNK
N_PAD = 10240
RPT = N_PAD // NS

_mesh = plsc.VectorSubcoreMesh(core_axis_name="c", subcore_axis_name="s")


@functools.partial(
    pl.kernel,
    mesh=_mesh,
    out_type=jax.ShapeDtypeStruct((NC, N_PAD), jnp.float32),
    scratch_types=[
        pltpu.VMEM_SHARED((N_PAD,), jnp.float32),
        pltpu.VMEM((CPT, CHUNK), jnp.int32),
        pltpu.VMEM((CHUNK,), jnp.float32),
    ],
)
def _deg_kernel(edges_hbm, zeros_hbm, out_hbm, acc, dst_v, ones_v):
    cid = lax.axis_index("c")
    sid = lax.axis_index("s")
    wid = cid * NS + sid
    pltpu.sync_copy(zeros_hbm, acc.at[pl.ds(sid * RPT, RPT)])
    pltpu.sync_copy(edges_hbm.at[1, wid], dst_v)
    for j in range(CHUNK // 16):
        ones_v[pl.ds(j * 16, 16)] = jnp.ones((16,), jnp.float32)
    plsc.subcore_barrier()

    def body(j, carry):
        pltpu.sync_copy(ones_v, acc.at[dst_v.at[j]], add=True)
        return carry

    lax.fori_loop(0, CPT, body, 0)
    plsc.subcore_barrier()
    pltpu.sync_copy(acc.at[pl.ds(sid * RPT, RPT)],
                    out_hbm.at[cid, pl.ds(sid * RPT, RPT)])


@functools.partial(
    pl.kernel,
    mesh=_mesh,
    out_type=jax.ShapeDtypeStruct((NC, N_PAD, D), jnp.float32),
    scratch_types=[
        pltpu.VMEM_SHARED((N_PAD, D), jnp.float32),
        pltpu.VMEM((2, SUP, CHUNK), jnp.int32),
        pltpu.VMEM((2, SUP, CHUNK), jnp.int32),
        pltpu.VMEM((2, CHUNK, D), jnp.float32),
        pltpu.SemaphoreType.DMA,
        pltpu.SemaphoreType.DMA,
        pltpu.SemaphoreType.DMA,
        pltpu.SemaphoreType.DMA,
        pltpu.SemaphoreType.DMA,
        pltpu.SemaphoreType.DMA,
    ],
)
def _agg_kernel(h_hbm, edges_hbm, zeros_hbm, out_hbm,
                acc, src_v, dst_v, rows_v, isem0, isem1, gsem0, gsem1,
                ssem0, ssem1):
    cid = lax.axis_index("c")
    sid = lax.axis_index("s")
    wid = cid * NS + sid
    isems = (isem0, isem1)
    gsems = (gsem0, gsem1)
    ssems = (ssem0, ssem1)

    def _idx_cps(s, b):
        return (pltpu.make_async_copy(
                    edges_hbm.at[0, wid, pl.ds(s * SUP, SUP)],
                    src_v.at[b], isems[b]),
                pltpu.make_async_copy(
                    edges_hbm.at[1, wid, pl.ds(s * SUP, SUP)],
                    dst_v.at[b], isems[b]))

    def _gather(p, b, c):
        return pltpu.make_async_copy(h_hbm.at[src_v.at[b, c]], rows_v.at[p],
                                     gsems[p])

    def _scatter(p, b, c):
        return pltpu.async_copy(rows_v.at[p], acc.at[dst_v.at[b, c]],
                                ssems[p], add=True)

    for cp in _idx_cps(0, 0):
        cp.start()
    pltpu.sync_copy(zeros_hbm, acc.at[pl.ds(sid * RPT, RPT), :])
    for cp in _idx_cps(0, 0):
        cp.wait()
    plsc.subcore_barrier()

    def body(g, carry):
        for b in range(2):
            s = 2 * g + b

            @pl.when(s + 1 < NSUP)
            def _(b=b, s=s):
                for cp in _idx_cps(s + 1, 1 - b):
                    cp.start()

            _gather(0, b, 0).start()
            _gather(1, b, 1).start()
            for c in range(SUP):
                p = c % 2
                _gather(p, b, c).wait()
                _scatter(p, b, c).wait()
                if c + 2 < SUP:
                    _gather(p, b, c + 2).start()

            @pl.when(s + 1 < NSUP)
            def _(b=b, s=s):
                for cp in _idx_cps(s + 1, 1 - b):
                    cp.wait()
        return carry

    lax.fori_loop(0, NSUP // 2, body, 0)
    plsc.subcore_barrier()
    pltpu.sync_copy(acc.at[pl.ds(sid * RPT, RPT), :],
                    out_hbm.at[cid, pl.ds(sid * RPT, RPT), :])


BLK = 2048
_GRID = N_PAD // BLK


def _mm_body(x_ref, w_ref, h_ref):
    h_ref[...] = jnp.dot(x_ref[...], w_ref[...],
                         preferred_element_type=jnp.float32)


def _scale_body(d0_ref, d1_ref, h_ref, hp_ref, dis_ref):
    deg = d0_ref[...] + d1_ref[...] + 1.0
    dis = lax.rsqrt(deg)
    dis_ref[...] = dis
    hp_ref[...] = dis * h_ref[...]


def _tc2_body(p0_ref, p1_ref, hp_ref, dis_ref, b_ref, w_ref, out_ref):
    dis = dis_ref[...]
    z = dis * (p0_ref[...] + p1_ref[...] + hp_ref[...]) + b_ref[...]
    out_ref[...] = dis * jnp.dot(z, w_ref[...],
                                 preferred_element_type=jnp.float32)


def _tc3_body(q0_ref, q1_ref, hp_ref, dis_ref, b_ref, out_ref):
    out_ref[...] = (dis_ref[...] * (q0_ref[...] + q1_ref[...] + hp_ref[...])
                    + b_ref[...])


_row_spec = pl.BlockSpec((BLK, D), lambda i: (i, 0))
_vec_spec = pl.BlockSpec((BLK, 1), lambda i: (i, 0))
_w_spec = pl.BlockSpec((D, D), lambda i: (0, 0))
_b_spec = pl.BlockSpec((1, D), lambda i: (0, 0))
_p1_spec = pl.BlockSpec((BLK, D), lambda i: (i + _GRID, 0))
_d1_spec = pl.BlockSpec((BLK, 1), lambda i: (i + _GRID, 0))


def _mm(xp, W1):
    return pl.pallas_call(
        _mm_body,
        grid=(_GRID,),
        in_specs=[_row_spec, _w_spec],
        out_specs=_row_spec,
        out_shape=jax.ShapeDtypeStruct((N_PAD, D), jnp.float32),
    )(xp, W1)


def _scale(degr, h1):
    return pl.pallas_call(
        _scale_body,
        grid=(_GRID,),
        in_specs=[_vec_spec, _d1_spec, _row_spec],
        out_specs=[_row_spec, _vec_spec],
        out_shape=[jax.ShapeDtypeStruct((N_PAD, D), jnp.float32),
                   jax.ShapeDtypeStruct((N_PAD, 1), jnp.float32)],
    )(degr, degr, h1)


def _tc2(Pr, hp, dis, b, W2):
    return pl.pallas_call(
        _tc2_body,
        grid=(_GRID,),
        in_specs=[_row_spec, _p1_spec, _row_spec, _vec_spec, _b_spec,
                  _w_spec],
        out_specs=_row_spec,
        out_shape=jax.ShapeDtypeStruct((N_PAD, D), jnp.float32),
    )(Pr, Pr, hp, dis, b, W2)


def _tc3(Qr, hp, dis, b):
    return pl.pallas_call(
        _tc3_body,
        grid=(_GRID,),
        in_specs=[_row_spec, _p1_spec, _row_spec, _vec_spec, _b_spec],
        out_specs=_row_spec,
        out_shape=jax.ShapeDtypeStruct((N_PAD, D), jnp.float32),
    )(Qr, Qr, hp, dis, b)


def kernel(x, edge_index, W1, b1, W2, b2):
    pad = E_PAD - E
    iota = jnp.arange(pad, dtype=jnp.int32)
    pad_block = jnp.stack([iota % N, N + iota % (N_PAD - N)])
    edges = jnp.concatenate([edge_index.astype(jnp.int32), pad_block],
                            axis=1).reshape(2, NW, CPT, CHUNK)
    zeros1 = jnp.zeros((RPT,), jnp.float32)
    zeros2 = jnp.zeros((RPT, D), jnp.float32)
    xp = jnp.concatenate([x, jnp.zeros((N_PAD - N, D), jnp.float32)])

    degp = _deg_kernel(edges, zeros1)
    h1 = _mm(xp, W1)
    h1p, dis = _scale(degp.reshape(NC * N_PAD, 1), h1)

    P = _agg_kernel(h1p, edges, zeros2)
    h2p = _tc2(P.reshape(NC * N_PAD, D), h1p, dis, b1.reshape(1, D), W2)
    Q = _agg_kernel(h2p, edges, zeros2)
    out = _tc3(Q.reshape(NC * N_PAD, D), h2p, dis, b2.reshape(1, D))
    return out[:N]


# speedup vs baseline: 1.2045x; 1.0084x over previous
import functools

import jax
import jax.numpy as jnp
from jax import lax
from jax.experimental import pallas as pl
from jax.experimental.pallas import tpu as pltpu, tpu_sc as plsc

N = 10000
D = 128
E = 320000

NC = 2
NS = 16
NW = NC * NS
CHUNK = 128
SUP = 8
NSUP = 10
CPT = SUP * NSUP
E_PAD = NW * CPT * CHUNK
N_PAD = 10240
RPT = N_PAD // NS

_mesh = plsc.VectorSubcoreMesh(core_axis_name="c", subcore_axis_name="s")


@functools.partial(
    pl.kernel,
    mesh=_mesh,
    out_type=jax.ShapeDtypeStruct((NC, N_PAD), jnp.float32),
    scratch_types=[
        pltpu.VMEM_SHARED((N_PAD,), jnp.float32),
        pltpu.VMEM((CPT, CHUNK), jnp.int32),
        pltpu.VMEM((CHUNK,), jnp.float32),
        pltpu.SemaphoreType.DMA,
        pltpu.SemaphoreType.DMA,
    ],
)
def _deg_kernel(edges_hbm, zeros_hbm, out_hbm, acc, dst_v, ones_v,
                ssem0, ssem1):
    cid = lax.axis_index("c")
    sid = lax.axis_index("s")
    wid = cid * NS + sid
    pltpu.sync_copy(zeros_hbm, acc.at[pl.ds(sid * RPT, RPT)])
    pltpu.sync_copy(edges_hbm.at[1, wid], dst_v)
    for j in range(CHUNK // 16):
        ones_v[pl.ds(j * 16, 16)] = jnp.ones((16,), jnp.float32)
    plsc.subcore_barrier()

    def body(g, carry):
        s0 = pltpu.async_copy(ones_v, acc.at[dst_v.at[2 * g]], ssem0,
                              add=True)
        s1 = pltpu.async_copy(ones_v, acc.at[dst_v.at[2 * g + 1]], ssem1,
                              add=True)
        s0.wait()
        s1.wait()
        return carry

    lax.fori_loop(0, CPT // 2, body, 0)
    plsc.subcore_barrier()
    pltpu.sync_copy(acc.at[pl.ds(sid * RPT, RPT)],
                    out_hbm.at[cid, pl.ds(sid * RPT, RPT)])


@functools.partial(
    pl.kernel,
    mesh=_mesh,
    out_type=jax.ShapeDtypeStruct((NC, N_PAD, D), jnp.float32),
    scratch_types=[
        pltpu.VMEM_SHARED((N_PAD, D), jnp.float32),
        pltpu.VMEM((2, SUP, CHUNK), jnp.int32),
        pltpu.VMEM((2, SUP, CHUNK), jnp.int32),
        pltpu.VMEM((2, CHUNK, D), jnp.float32),
        pltpu.SemaphoreType.DMA,
        pltpu.SemaphoreType.DMA,
        pltpu.SemaphoreType.DMA,
        pltpu.SemaphoreType.DMA,
        pltpu.SemaphoreType.DMA,
        pltpu.SemaphoreType.DMA,
    ],
)
def _agg_kernel(h_hbm, edges_hbm, zeros_hbm, out_hbm,
                acc, src_v, dst_v, rows_v, isem0, isem1, gsem0, gsem1,
                ssem0, ssem1):
    cid = lax.axis_index("c")
    sid = lax.axis_index("s")
    wid = cid * NS + sid
    isems = (isem0, isem1)
    gsems = (gsem0, gsem1)
    ssems = (ssem0, ssem1)

    def _idx_cps(s, b):
        return (pltpu.make_async_copy(
                    edges_hbm.at[0, wid, pl.ds(s * SUP, SUP)],
                    src_v.at[b], isems[b]),
                pltpu.make_async_copy(
                    edges_hbm.at[1, wid, pl.ds(s * SUP, SUP)],
                    dst_v.at[b], isems[b]))

    def _gather(p, b, c):
        return pltpu.make_async_copy(h_hbm.at[src_v.at[b, c]], rows_v.at[p],
                                     gsems[p])

    def _scatter(p, b, c):
        return pltpu.async_copy(rows_v.at[p], acc.at[dst_v.at[b, c]],
                                ssems[p], add=True)

    for cp in _idx_cps(0, 0):
        cp.start()
    pltpu.sync_copy(zeros_hbm, acc.at[pl.ds(sid * RPT, RPT), :])
    for cp in _idx_cps(0, 0):
        cp.wait()
    plsc.subcore_barrier()

    def body(g, carry):
        for b in range(2):
            s = 2 * g + b

            @pl.when(s + 1 < NSUP)
            def _(b=b, s=s):
                for cp in _idx_cps(s + 1, 1 - b):
                    cp.start()

            _gather(0, b, 0).start()
            _gather(1, b, 1).start()
            for c in range(SUP):
                p = c % 2
                _gather(p, b, c).wait()
                _scatter(p, b, c).wait()
                if c + 2 < SUP:
                    _gather(p, b, c + 2).start()

            @pl.when(s + 1 < NSUP)
            def _(b=b, s=s):
                for cp in _idx_cps(s + 1, 1 - b):
                    cp.wait()
        return carry

    lax.fori_loop(0, NSUP // 2, body, 0)
    plsc.subcore_barrier()
    pltpu.sync_copy(acc.at[pl.ds(sid * RPT, RPT), :],
                    out_hbm.at[cid, pl.ds(sid * RPT, RPT), :])


BLK = 2048
_GRID = N_PAD // BLK


def _mm_body(x_ref, w_ref, h_ref):
    h_ref[...] = jnp.dot(x_ref[...], w_ref[...],
                         preferred_element_type=jnp.float32)


def _scale_body(d0_ref, d1_ref, h_ref, hp_ref, dis_ref):
    deg = d0_ref[...] + d1_ref[...] + 1.0
    dis = lax.rsqrt(deg)
    dis_ref[...] = dis
    hp_ref[...] = dis * h_ref[...]


def _tc2_body(p0_ref, p1_ref, hp_ref, dis_ref, b_ref, w_ref, out_ref):
    dis = dis_ref[...]
    z = dis * (p0_ref[...] + p1_ref[...] + hp_ref[...]) + b_ref[...]
    out_ref[...] = dis * jnp.dot(z, w_ref[...],
                                 preferred_element_type=jnp.float32)


def _tc3_body(q0_ref, q1_ref, hp_ref, dis_ref, b_ref, out_ref):
    out_ref[...] = (dis_ref[...] * (q0_ref[...] + q1_ref[...] + hp_ref[...])
                    + b_ref[...])


_row_spec = pl.BlockSpec((BLK, D), lambda i: (i, 0))
_vec_spec = pl.BlockSpec((BLK, 1), lambda i: (i, 0))
_w_spec = pl.BlockSpec((D, D), lambda i: (0, 0))
_b_spec = pl.BlockSpec((1, D), lambda i: (0, 0))
_p1_spec = pl.BlockSpec((BLK, D), lambda i: (i + _GRID, 0))
_d1_spec = pl.BlockSpec((BLK, 1), lambda i: (i + _GRID, 0))


def _mm(xp, W1):
    return pl.pallas_call(
        _mm_body,
        grid=(_GRID,),
        in_specs=[_row_spec, _w_spec],
        out_specs=_row_spec,
        out_shape=jax.ShapeDtypeStruct((N_PAD, D), jnp.float32),
    )(xp, W1)


def _scale(degr, h1):
    return pl.pallas_call(
        _scale_body,
        grid=(_GRID,),
        in_specs=[_vec_spec, _d1_spec, _row_spec],
        out_specs=[_row_spec, _vec_spec],
        out_shape=[jax.ShapeDtypeStruct((N_PAD, D), jnp.float32),
                   jax.ShapeDtypeStruct((N_PAD, 1), jnp.float32)],
    )(degr, degr, h1)


def _tc2(Pr, hp, dis, b, W2):
    return pl.pallas_call(
        _tc2_body,
        grid=(_GRID,),
        in_specs=[_row_spec, _p1_spec, _row_spec, _vec_spec, _b_spec,
                  _w_spec],
        out_specs=_row_spec,
        out_shape=jax.ShapeDtypeStruct((N_PAD, D), jnp.float32),
    )(Pr, Pr, hp, dis, b, W2)


def _tc3(Qr, hp, dis, b):
    return pl.pallas_call(
        _tc3_body,
        grid=(_GRID,),
        in_specs=[_row_spec, _p1_spec, _row_spec, _vec_spec, _b_spec],
        out_specs=_row_spec,
        out_shape=jax.ShapeDtypeStruct((N_PAD, D), jnp.float32),
    )(Qr, Qr, hp, dis, b)


def kernel(x, edge_index, W1, b1, W2, b2):
    pad = E_PAD - E
    iota = jnp.arange(pad, dtype=jnp.int32)
    pad_block = jnp.stack([iota % N, N + iota % (N_PAD - N)])
    edges = jnp.concatenate([edge_index.astype(jnp.int32), pad_block],
                            axis=1).reshape(2, NW, CPT, CHUNK)
    zeros1 = jnp.zeros((RPT,), jnp.float32)
    zeros2 = jnp.zeros((RPT, D), jnp.float32)
    xp = jnp.concatenate([x, jnp.zeros((N_PAD - N, D), jnp.float32)])

    degp = _deg_kernel(edges, zeros1)
    h1 = _mm(xp, W1)
    h1p, dis = _scale(degp.reshape(NC * N_PAD, 1), h1)

    P = _agg_kernel(h1p, edges, zeros2)
    h2p = _tc2(P.reshape(NC * N_PAD, D), h1p, dis, b1.reshape(1, D), W2)
    Q = _agg_kernel(h2p, edges, zeros2)
    out = _tc3(Q.reshape(NC * N_PAD, D), h2p, dis, b2.reshape(1, D))
    return out[:N]


# speedup vs baseline: 1.2047x; 1.0002x over previous
import functools

import jax
import jax.numpy as jnp
from jax import lax
from jax.experimental import pallas as pl
from jax.experimental.pallas import tpu as pltpu, tpu_sc as plsc

N = 10000
D = 128
E = 320000

NC = 2
NS = 16
NW = NC * NS
CHUNK = 128
SUP = 8
NSUP = 10
CPT = SUP * NSUP
E_PAD = NW * CPT * CHUNK
N_PAD = 10240
RPT = N_PAD // NS

_mesh = plsc.VectorSubcoreMesh(core_axis_name="c", subcore_axis_name="s")


@functools.partial(
    pl.kernel,
    mesh=_mesh,
    out_type=jax.ShapeDtypeStruct((NC, N_PAD), jnp.float32),
    scratch_types=[
        pltpu.VMEM_SHARED((N_PAD,), jnp.float32),
        pltpu.VMEM((CPT, CHUNK), jnp.int32),
        pltpu.VMEM((CHUNK,), jnp.float32),
        pltpu.SemaphoreType.DMA,
        pltpu.SemaphoreType.DMA,
    ],
)
def _deg_kernel(edges_hbm, zeros_hbm, out_hbm, acc, dst_v, ones_v,
                ssem0, ssem1):
    cid = lax.axis_index("c")
    sid = lax.axis_index("s")
    wid = cid * NS + sid
    pltpu.sync_copy(zeros_hbm, acc.at[pl.ds(sid * RPT, RPT)])
    pltpu.sync_copy(edges_hbm.at[1, wid], dst_v)
    for j in range(CHUNK // 16):
        ones_v[pl.ds(j * 16, 16)] = jnp.ones((16,), jnp.float32)
    plsc.subcore_barrier()

    def body(g, carry):
        s0 = pltpu.async_copy(ones_v, acc.at[dst_v.at[2 * g]], ssem0,
                              add=True)
        s1 = pltpu.async_copy(ones_v, acc.at[dst_v.at[2 * g + 1]], ssem1,
                              add=True)
        s0.wait()
        s1.wait()
        return carry

    lax.fori_loop(0, CPT // 2, body, 0)
    plsc.subcore_barrier()
    pltpu.sync_copy(acc.at[pl.ds(sid * RPT, RPT)],
                    out_hbm.at[cid, pl.ds(sid * RPT, RPT)])


@functools.partial(
    pl.kernel,
    mesh=_mesh,
    out_type=jax.ShapeDtypeStruct((NC, N_PAD, D), jnp.float32),
    scratch_types=[
        pltpu.VMEM_SHARED((N_PAD, D), jnp.float32),
        pltpu.VMEM((2, SUP, CHUNK), jnp.int32),
        pltpu.VMEM((2, SUP, CHUNK), jnp.int32),
        pltpu.VMEM((2, CHUNK, D), jnp.float32),
        pltpu.SemaphoreType.DMA,
        pltpu.SemaphoreType.DMA,
        pltpu.SemaphoreType.DMA,
        pltpu.SemaphoreType.DMA,
        pltpu.SemaphoreType.DMA,
        pltpu.SemaphoreType.DMA,
    ],
)
def _agg_kernel(h_hbm, edges_hbm, zeros_hbm, out_hbm,
                acc, src_v, dst_v, rows_v, isem0, isem1, gsem0, gsem1,
                ssem0, ssem1):
    cid = lax.axis_index("c")
    sid = lax.axis_index("s")
    wid = cid * NS + sid
    isems = (isem0, isem1)
    gsems = (gsem0, gsem1)
    ssems = (ssem0, ssem1)

    def _idx_cps(s, b):
        return (pltpu.make_async_copy(
                    edges_hbm.at[0, wid, pl.ds(s * SUP, SUP)],
                    src_v.at[b], isems[b]),
                pltpu.make_async_copy(
                    edges_hbm.at[1, wid, pl.ds(s * SUP, SUP)],
                    dst_v.at[b], isems[b]))

    def _gather(p, b, c):
        return pltpu.make_async_copy(h_hbm.at[src_v.at[b, c]], rows_v.at[p],
                                     gsems[p])

    def _scatter(p, b, c):
        return pltpu.async_copy(rows_v.at[p], acc.at[dst_v.at[b, c]],
                                ssems[p], add=True)

    for cp in _idx_cps(0, 0):
        cp.start()
    pltpu.sync_copy(zeros_hbm, acc.at[pl.ds(sid * RPT, RPT), :])
    for cp in _idx_cps(0, 0):
        cp.wait()
    plsc.subcore_barrier()

    def body(g, carry):
        for b in range(2):
            s = 2 * g + b

            @pl.when(s + 1 < NSUP)
            def _(b=b, s=s):
                for cp in _idx_cps(s + 1, 1 - b):
                    cp.start()

            _gather(0, b, 0).start()
            _gather(1, b, 1).start()
            for c in range(SUP):
                p = c % 2
                _gather(p, b, c).wait()
                _scatter(p, b, c).wait()
                if c + 2 < SUP:
                    _gather(p, b, c + 2).start()

            @pl.when(s + 1 < NSUP)
            def _(b=b, s=s):
                for cp in _idx_cps(s + 1, 1 - b):
                    cp.wait()
        return carry

    lax.fori_loop(0, NSUP // 2, body, 0)
    plsc.subcore_barrier()
    pltpu.sync_copy(acc.at[pl.ds(sid * RPT, RPT), :],
                    out_hbm.at[cid, pl.ds(sid * RPT, RPT), :])


BLK = 2048
_GRID = N_PAD // BLK


def _mm_body(x_ref, w_ref, h_ref):
    h_ref[...] = jnp.dot(x_ref[...], w_ref[...],
                         preferred_element_type=jnp.float32)


def _scale_body(d0_ref, d1_ref, h_ref, hp_ref, dis_ref):
    deg = d0_ref[...] + d1_ref[...] + 1.0
    dis = lax.rsqrt(deg)
    dis_ref[...] = dis
    hp_ref[...] = dis * h_ref[...]


def _tc2_body(p0_ref, p1_ref, hp_ref, dis_ref, b_ref, w_ref, out_ref):
    dis = dis_ref[...]
    z = dis * (p0_ref[...] + p1_ref[...] + hp_ref[...]) + b_ref[...]
    out_ref[...] = dis * jnp.dot(z, w_ref[...],
                                 preferred_element_type=jnp.float32)


def _tc3_body(q0_ref, q1_ref, hp_ref, dis_ref, b_ref, out_ref):
    out_ref[...] = (dis_ref[...] * (q0_ref[...] + q1_ref[...] + hp_ref[...])
                    + b_ref[...])


_row_spec = pl.BlockSpec((BLK, D), lambda i: (i, 0))
_vec_spec = pl.BlockSpec((BLK, 1), lambda i: (i, 0))
_w_spec = pl.BlockSpec((D, D), lambda i: (0, 0))
_b_spec = pl.BlockSpec((1, D), lambda i: (0, 0))
_p1_spec = pl.BlockSpec((BLK, D), lambda i: (i + _GRID, 0))
_d1_spec = pl.BlockSpec((BLK, 1), lambda i: (i + _GRID, 0))


_MMBLK = 2000


def _mm(x, W1):
    return pl.pallas_call(
        _mm_body,
        grid=(N // _MMBLK,),
        in_specs=[pl.BlockSpec((_MMBLK, D), lambda i: (i, 0)), _w_spec],
        out_specs=pl.BlockSpec((_MMBLK, D), lambda i: (i, 0)),
        out_shape=jax.ShapeDtypeStruct((N_PAD, D), jnp.float32),
    )(x, W1)


def _scale(degr, h1):
    return pl.pallas_call(
        _scale_body,
        grid=(_GRID,),
        in_specs=[_vec_spec, _d1_spec, _row_spec],
        out_specs=[_row_spec, _vec_spec],
        out_shape=[jax.ShapeDtypeStruct((N_PAD, D), jnp.float32),
                   jax.ShapeDtypeStruct((N_PAD, 1), jnp.float32)],
    )(degr, degr, h1)


def _tc2(Pr, hp, dis, b, W2):
    return pl.pallas_call(
        _tc2_body,
        grid=(_GRID,),
        in_specs=[_row_spec, _p1_spec, _row_spec, _vec_spec, _b_spec,
                  _w_spec],
        out_specs=_row_spec,
        out_shape=jax.ShapeDtypeStruct((N_PAD, D), jnp.float32),
    )(Pr, Pr, hp, dis, b, W2)


def _tc3(Qr, hp, dis, b):
    return pl.pallas_call(
        _tc3_body,
        grid=(_GRID,),
        in_specs=[_row_spec, _p1_spec, _row_spec, _vec_spec, _b_spec],
        out_specs=_row_spec,
        out_shape=jax.ShapeDtypeStruct((N_PAD, D), jnp.float32),
    )(Qr, Qr, hp, dis, b)


def kernel(x, edge_index, W1, b1, W2, b2):
    pad = E_PAD - E
    iota = jnp.arange(pad, dtype=jnp.int32)
    pad_block = jnp.stack([iota % N, N + iota % (N_PAD - N)])
    edges = jnp.concatenate([edge_index.astype(jnp.int32), pad_block],
                            axis=1).reshape(2, NW, CPT, CHUNK)
    zeros1 = jnp.zeros((RPT,), jnp.float32)
    zeros2 = jnp.zeros((RPT, D), jnp.float32)

    degp = _deg_kernel(edges, zeros1)
    h1 = _mm(x, W1)
    h1p, dis = _scale(degp.reshape(NC * N_PAD, 1), h1)

    P = _agg_kernel(h1p, edges, zeros2)
    h2p = _tc2(P.reshape(NC * N_PAD, D), h1p, dis, b1.reshape(1, D), W2)
    Q = _agg_kernel(h2p, edges, zeros2)
    out = _tc3(Q.reshape(NC * N_PAD, D), h2p, dis, b2.reshape(1, D))
    return out[:N]


# speedup vs baseline: 1.2107x; 1.0050x over previous
import functools

import jax
import jax.numpy as jnp
from jax import lax
from jax.experimental import pallas as pl
from jax.experimental.pallas import tpu as pltpu, tpu_sc as plsc

N = 10000
D = 128
E = 320000

NC = 2
NS = 16
NW = NC * NS
CHUNK = 128
SUP = 8
NSUP = 10
CPT = SUP * NSUP
E_PAD = NW * CPT * CHUNK
N_PAD = 10240
RPT = N_PAD // NS

_mesh = plsc.VectorSubcoreMesh(core_axis_name="c", subcore_axis_name="s")


@functools.partial(
    pl.kernel,
    mesh=_mesh,
    out_type=jax.ShapeDtypeStruct((NC, N_PAD), jnp.float32),
    scratch_types=[
        pltpu.VMEM_SHARED((N_PAD,), jnp.float32),
        pltpu.VMEM((CPT, CHUNK), jnp.int32),
        pltpu.VMEM((CHUNK,), jnp.float32),
        pltpu.SemaphoreType.DMA,
        pltpu.SemaphoreType.DMA,
    ],
)
def _deg_kernel(edges_hbm, zeros_hbm, out_hbm, acc, dst_v, ones_v,
                ssem0, ssem1):
    cid = lax.axis_index("c")
    sid = lax.axis_index("s")
    wid = cid * NS + sid
    pltpu.sync_copy(zeros_hbm, acc.at[pl.ds(sid * RPT, RPT)])
    pltpu.sync_copy(edges_hbm.at[1, wid], dst_v)
    for j in range(CHUNK // 16):
        ones_v[pl.ds(j * 16, 16)] = jnp.ones((16,), jnp.float32)
    plsc.subcore_barrier()

    def body(g, carry):
        s0 = pltpu.async_copy(ones_v, acc.at[dst_v.at[2 * g]], ssem0,
                              add=True)
        s1 = pltpu.async_copy(ones_v, acc.at[dst_v.at[2 * g + 1]], ssem1,
                              add=True)
        s0.wait()
        s1.wait()
        return carry

    lax.fori_loop(0, CPT // 2, body, 0)
    plsc.subcore_barrier()
    pltpu.sync_copy(acc.at[pl.ds(sid * RPT, RPT)],
                    out_hbm.at[cid, pl.ds(sid * RPT, RPT)])


@functools.partial(
    pl.kernel,
    mesh=_mesh,
    out_type=jax.ShapeDtypeStruct((NC, N_PAD, D), jnp.float32),
    scratch_types=[
        pltpu.VMEM_SHARED((N_PAD, D), jnp.float32),
        pltpu.VMEM((2, SUP, CHUNK), jnp.int32),
        pltpu.VMEM((2, SUP, CHUNK), jnp.int32),
        pltpu.VMEM((2, CHUNK, D), jnp.float32),
        pltpu.SemaphoreType.DMA,
        pltpu.SemaphoreType.DMA,
        pltpu.SemaphoreType.DMA,
        pltpu.SemaphoreType.DMA,
        pltpu.SemaphoreType.DMA,
        pltpu.SemaphoreType.DMA,
    ],
)
def _agg_kernel(h_hbm, edges_hbm, zeros_hbm, out_hbm,
                acc, src_v, dst_v, rows_v, isem0, isem1, gsem0, gsem1,
                ssem0, ssem1):
    cid = lax.axis_index("c")
    sid = lax.axis_index("s")
    wid = cid * NS + sid
    isems = (isem0, isem1)
    gsems = (gsem0, gsem1)
    ssems = (ssem0, ssem1)

    def _idx_cps(s, b):
        return (pltpu.make_async_copy(
                    edges_hbm.at[0, wid, pl.ds(s * SUP, SUP)],
                    src_v.at[b], isems[b]),
                pltpu.make_async_copy(
                    edges_hbm.at[1, wid, pl.ds(s * SUP, SUP)],
                    dst_v.at[b], isems[b]))

    def _gather(p, b, c):
        return pltpu.make_async_copy(h_hbm.at[src_v.at[b, c]], rows_v.at[p],
                                     gsems[p])

    def _scatter(p, b, c):
        return pltpu.async_copy(rows_v.at[p], acc.at[dst_v.at[b, c]],
                                ssems[p], add=True)

    for cp in _idx_cps(0, 0):
        cp.start()
    pltpu.sync_copy(zeros_hbm, acc.at[pl.ds(sid * RPT, RPT), :])
    for cp in _idx_cps(0, 0):
        cp.wait()
    plsc.subcore_barrier()

    def body(g, carry):
        for b in range(2):
            s = 2 * g + b

            @pl.when(s + 1 < NSUP)
            def _(b=b, s=s):
                for cp in _idx_cps(s + 1, 1 - b):
                    cp.start()

            _gather(0, b, 0).start()
            _gather(1, b, 1).start()
            for c in range(SUP):
                p = c % 2
                _gather(p, b, c).wait()
                _scatter(p, b, c).wait()
                if c + 2 < SUP:
                    _gather(p, b, c + 2).start()

            @pl.when(s + 1 < NSUP)
            def _(b=b, s=s):
                for cp in _idx_cps(s + 1, 1 - b):
                    cp.wait()
        return carry

    lax.fori_loop(0, NSUP // 2, body, 0)
    plsc.subcore_barrier()
    pltpu.sync_copy(acc.at[pl.ds(sid * RPT, RPT), :],
                    out_hbm.at[cid, pl.ds(sid * RPT, RPT), :])


BLK = 2048
_GRID = N_PAD // BLK


def _mm_body(x_ref, w_ref, h_ref):
    h_ref[...] = jnp.dot(x_ref[...], w_ref[...],
                         preferred_element_type=jnp.float32)


def _scale_body(d0_ref, d1_ref, x_ref, w_ref, hp_ref, dis_ref):
    deg = d0_ref[...] + d1_ref[...] + 1.0
    dis = lax.rsqrt(deg)
    dis_ref[...] = dis
    hp_ref[...] = dis * jnp.dot(x_ref[...], w_ref[...],
                                preferred_element_type=jnp.float32)


def _tc2_body(p0_ref, p1_ref, hp_ref, dis_ref, b_ref, w_ref, out_ref):
    dis = dis_ref[...]
    z = dis * (p0_ref[...] + p1_ref[...] + hp_ref[...]) + b_ref[...]
    out_ref[...] = dis * jnp.dot(z, w_ref[...],
                                 preferred_element_type=jnp.float32)


def _tc3_body(q0_ref, q1_ref, hp_ref, dis_ref, b_ref, out_ref):
    out_ref[...] = (dis_ref[...] * (q0_ref[...] + q1_ref[...] + hp_ref[...])
                    + b_ref[...])


_row_spec = pl.BlockSpec((BLK, D), lambda i: (i, 0))
_vec_spec = pl.BlockSpec((BLK, 1), lambda i: (i, 0))
_w_spec = pl.BlockSpec((D, D), lambda i: (0, 0))
_b_spec = pl.BlockSpec((1, D), lambda i: (0, 0))
_p1_spec = pl.BlockSpec((BLK, D), lambda i: (i + _GRID, 0))
_d1_spec = pl.BlockSpec((BLK, 1), lambda i: (i + _GRID, 0))


_MMBLK = 2000


def _mm(x, W1):
    return pl.pallas_call(
        _mm_body,
        grid=(N // _MMBLK,),
        in_specs=[pl.BlockSpec((_MMBLK, D), lambda i: (i, 0)), _w_spec],
        out_specs=pl.BlockSpec((_MMBLK, D), lambda i: (i, 0)),
        out_shape=jax.ShapeDtypeStruct((N_PAD, D), jnp.float32),
    )(x, W1)


_XBLK = 2000


def _scale(d0, d1, x, W1):
    return pl.pallas_call(
        _scale_body,
        grid=(N // _XBLK,),
        in_specs=[pl.BlockSpec((_XBLK, 1), lambda i: (i, 0)),
                  pl.BlockSpec((_XBLK, 1), lambda i: (i, 0)),
                  pl.BlockSpec((_XBLK, D), lambda i: (i, 0)),
                  _w_spec],
        out_specs=[pl.BlockSpec((_XBLK, D), lambda i: (i, 0)),
                   pl.BlockSpec((_XBLK, 1), lambda i: (i, 0))],
        out_shape=[jax.ShapeDtypeStruct((N_PAD, D), jnp.float32),
                   jax.ShapeDtypeStruct((N_PAD, 1), jnp.float32)],
    )(d0, d1, x, W1)


def _tc2(Pr, hp, dis, b, W2):
    return pl.pallas_call(
        _tc2_body,
        grid=(_GRID,),
        in_specs=[_row_spec, _p1_spec, _row_spec, _vec_spec, _b_spec,
                  _w_spec],
        out_specs=_row_spec,
        out_shape=jax.ShapeDtypeStruct((N_PAD, D), jnp.float32),
    )(Pr, Pr, hp, dis, b, W2)


def _tc3(Qr, hp, dis, b):
    return pl.pallas_call(
        _tc3_body,
        grid=(_GRID,),
        in_specs=[_row_spec, _p1_spec, _row_spec, _vec_spec, _b_spec],
        out_specs=_row_spec,
        out_shape=jax.ShapeDtypeStruct((N_PAD, D), jnp.float32),
    )(Qr, Qr, hp, dis, b)


def kernel(x, edge_index, W1, b1, W2, b2):
    pad = E_PAD - E
    iota = jnp.arange(pad, dtype=jnp.int32)
    pad_block = jnp.stack([iota % N, N + iota % (N_PAD - N)])
    edges = jnp.concatenate([edge_index.astype(jnp.int32), pad_block],
                            axis=1).reshape(2, NW, CPT, CHUNK)
    zeros1 = jnp.zeros((RPT,), jnp.float32)
    zeros2 = jnp.zeros((RPT, D), jnp.float32)

    degp = _deg_kernel(edges, zeros1)
    h1p, dis = _scale(degp[0, :, None], degp[1, :, None], x, W1)

    P = _agg_kernel(h1p, edges, zeros2)
    h2p = _tc2(P.reshape(NC * N_PAD, D), h1p, dis, b1.reshape(1, D), W2)
    Q = _agg_kernel(h2p, edges, zeros2)
    out = _tc3(Q.reshape(NC * N_PAD, D), h2p, dis, b2.reshape(1, D))
    return out[:N]


# speedup vs baseline: 1.2120x; 1.0011x over previous
import functools

import jax
import jax.numpy as jnp
from jax import lax
from jax.experimental import pallas as pl
from jax.experimental.pallas import tpu as pltpu, tpu_sc as plsc

N = 10000
D = 128
E = 320000

NC = 2
NS = 16
NW = NC * NS
CHUNK = 128
SUP = 8
NSUP = 10
CPT = SUP * NSUP
E_PAD = NW * CPT * CHUNK
N_PAD = 10240
RPT = N_PAD // NS

_mesh = plsc.VectorSubcoreMesh(core_axis_name="c", subcore_axis_name="s")


@functools.partial(
    pl.kernel,
    mesh=_mesh,
    out_type=jax.ShapeDtypeStruct((NC, N_PAD), jnp.float32),
    scratch_types=[
        pltpu.VMEM_SHARED((N_PAD,), jnp.float32),
        pltpu.VMEM((CPT, CHUNK), jnp.int32),
        pltpu.VMEM((CHUNK,), jnp.float32),
        pltpu.SemaphoreType.DMA,
        pltpu.SemaphoreType.DMA,
    ],
)
def _deg_kernel(edges_hbm, zeros_hbm, out_hbm, acc, dst_v, ones_v,
                ssem0, ssem1):
    cid = lax.axis_index("c")
    sid = lax.axis_index("s")
    wid = cid * NS + sid
    pltpu.sync_copy(zeros_hbm, acc.at[pl.ds(sid * RPT, RPT)])
    pltpu.sync_copy(edges_hbm.at[1, wid], dst_v)
    for j in range(CHUNK // 16):
        ones_v[pl.ds(j * 16, 16)] = jnp.ones((16,), jnp.float32)
    plsc.subcore_barrier()

    def body(g, carry):
        s0 = pltpu.async_copy(ones_v, acc.at[dst_v.at[2 * g]], ssem0,
                              add=True)
        s1 = pltpu.async_copy(ones_v, acc.at[dst_v.at[2 * g + 1]], ssem1,
                              add=True)
        s0.wait()
        s1.wait()
        return carry

    lax.fori_loop(0, CPT // 2, body, 0)
    plsc.subcore_barrier()
    pltpu.sync_copy(acc.at[pl.ds(sid * RPT, RPT)],
                    out_hbm.at[cid, pl.ds(sid * RPT, RPT)])


@functools.partial(
    pl.kernel,
    mesh=_mesh,
    out_type=jax.ShapeDtypeStruct((NC, N_PAD, D), jnp.float32),
    scratch_types=[
        pltpu.VMEM_SHARED((N_PAD, D), jnp.float32),
        pltpu.VMEM((2, SUP, CHUNK), jnp.int32),
        pltpu.VMEM((2, SUP, CHUNK), jnp.int32),
        pltpu.VMEM((2, CHUNK, D), jnp.float32),
        pltpu.SemaphoreType.DMA,
        pltpu.SemaphoreType.DMA,
        pltpu.SemaphoreType.DMA,
        pltpu.SemaphoreType.DMA,
        pltpu.SemaphoreType.DMA,
        pltpu.SemaphoreType.DMA,
    ],
)
def _agg_kernel(h_hbm, edges_hbm, zeros_hbm, out_hbm,
                acc, src_v, dst_v, rows_v, isem0, isem1, gsem0, gsem1,
                ssem0, ssem1):
    cid = lax.axis_index("c")
    sid = lax.axis_index("s")
    wid = cid * NS + sid
    isems = (isem0, isem1)
    gsems = (gsem0, gsem1)
    ssems = (ssem0, ssem1)

    def _idx_cps(s, b):
        return (pltpu.make_async_copy(
                    edges_hbm.at[0, wid, pl.ds(s * SUP, SUP)],
                    src_v.at[b], isems[b]),
                pltpu.make_async_copy(
                    edges_hbm.at[1, wid, pl.ds(s * SUP, SUP)],
                    dst_v.at[b], isems[b]))

    def _gather(p, b, c):
        return pltpu.make_async_copy(h_hbm.at[src_v.at[b, c]], rows_v.at[p],
                                     gsems[p])

    def _scatter(p, b, c):
        return pltpu.async_copy(rows_v.at[p], acc.at[dst_v.at[b, c]],
                                ssems[p], add=True)

    for cp in _idx_cps(0, 0):
        cp.start()
    pltpu.sync_copy(zeros_hbm, acc.at[pl.ds(sid * RPT, RPT), :])
    for cp in _idx_cps(0, 0):
        cp.wait()
    plsc.subcore_barrier()

    def body(g, carry):
        for b in range(2):
            s = 2 * g + b

            @pl.when(s + 1 < NSUP)
            def _(b=b, s=s):
                for cp in _idx_cps(s + 1, 1 - b):
                    cp.start()

            _gather(0, b, 0).start()
            _gather(1, b, 1).start()
            for c in range(SUP):
                p = c % 2
                _gather(p, b, c).wait()
                _scatter(p, b, c).wait()
                if c + 2 < SUP:
                    _gather(p, b, c + 2).start()

            @pl.when(s + 1 < NSUP)
            def _(b=b, s=s):
                for cp in _idx_cps(s + 1, 1 - b):
                    cp.wait()
        return carry

    lax.fori_loop(0, NSUP // 2, body, 0)
    plsc.subcore_barrier()
    pltpu.sync_copy(acc.at[pl.ds(sid * RPT, RPT), :],
                    out_hbm.at[cid, pl.ds(sid * RPT, RPT), :])


BLK = 2048
_GRID = N_PAD // BLK


def _scale_body(d0_ref, d1_ref, x_ref, w_ref, hp_ref, dis_ref):
    deg = d0_ref[...] + d1_ref[...] + 1.0
    dis = lax.rsqrt(deg)
    dis_ref[...] = dis
    hp_ref[...] = dis * jnp.dot(x_ref[...], w_ref[...],
                                preferred_element_type=jnp.float32)


def _tc2_body(p0_ref, p1_ref, hp_ref, dis_ref, b_ref, w_ref, out_ref):
    dis = dis_ref[...]
    z = dis * (p0_ref[...] + p1_ref[...] + hp_ref[...]) + b_ref[...]
    out_ref[...] = dis * jnp.dot(z, w_ref[...],
                                 preferred_element_type=jnp.float32)


def _tc3_body(q0_ref, q1_ref, hp_ref, dis_ref, b_ref, out_ref):
    out_ref[...] = (dis_ref[...] * (q0_ref[...] + q1_ref[...] + hp_ref[...])
                    + b_ref[...])


_row_spec = pl.BlockSpec((BLK, D), lambda i: (i, 0))
_vec_spec = pl.BlockSpec((BLK, 1), lambda i: (i, 0))
_w_spec = pl.BlockSpec((D, D), lambda i: (0, 0))
_b_spec = pl.BlockSpec((1, D), lambda i: (0, 0))
_p1_spec = pl.BlockSpec((BLK, D), lambda i: (i + _GRID, 0))
_d1_spec = pl.BlockSpec((BLK, 1), lambda i: (i + _GRID, 0))


_XBLK = 2000


def _scale(d0, d1, x, W1):
    return pl.pallas_call(
        _scale_body,
        grid=(N // _XBLK,),
        in_specs=[pl.BlockSpec((_XBLK, 1), lambda i: (i, 0)),
                  pl.BlockSpec((_XBLK, 1), lambda i: (i, 0)),
                  pl.BlockSpec((_XBLK, D), lambda i: (i, 0)),
                  _w_spec],
        out_specs=[pl.BlockSpec((_XBLK, D), lambda i: (i, 0)),
                   pl.BlockSpec((_XBLK, 1), lambda i: (i, 0))],
        out_shape=[jax.ShapeDtypeStruct((N_PAD, D), jnp.float32),
                   jax.ShapeDtypeStruct((N_PAD, 1), jnp.float32)],
    )(d0, d1, x, W1)


def _tc2(Pr, hp, dis, b, W2):
    return pl.pallas_call(
        _tc2_body,
        grid=(_GRID,),
        in_specs=[_row_spec, _p1_spec, _row_spec, _vec_spec, _b_spec,
                  _w_spec],
        out_specs=_row_spec,
        out_shape=jax.ShapeDtypeStruct((N_PAD, D), jnp.float32),
    )(Pr, Pr, hp, dis, b, W2)


def _tc3(Qr, hp, dis, b):
    return pl.pallas_call(
        _tc3_body,
        grid=(_GRID,),
        in_specs=[_row_spec, _p1_spec, _row_spec, _vec_spec, _b_spec],
        out_specs=_row_spec,
        out_shape=jax.ShapeDtypeStruct((N_PAD, D), jnp.float32),
    )(Qr, Qr, hp, dis, b)


def kernel(x, edge_index, W1, b1, W2, b2):
    pad = E_PAD - E
    iota = jnp.arange(pad, dtype=jnp.int32)
    pad_block = jnp.stack([iota % N, N + iota % (N_PAD - N)])
    edges = jnp.concatenate([edge_index.astype(jnp.int32), pad_block],
                            axis=1).reshape(2, NW, CPT, CHUNK)
    zeros1 = jnp.zeros((RPT,), jnp.float32)
    zeros2 = jnp.zeros((RPT, D), jnp.float32)

    degp = _deg_kernel(edges, zeros1)
    h1p, dis = _scale(degp[0, :, None], degp[1, :, None], x, W1)

    P = _agg_kernel(h1p, edges, zeros2)
    h2p = _tc2(P.reshape(NC * N_PAD, D), h1p, dis, b1.reshape(1, D), W2)
    Q = _agg_kernel(h2p, edges, zeros2)
    out = _tc3(Q.reshape(NC * N_PAD, D), h2p, dis, b2.reshape(1, D))
    return out[:N]
